# baseline (speedup 1.0000x reference)
import numpy as np
import jax
import jax.numpy as jnp
from jax.experimental import pallas as pl

_TBLK = 2048
_NEG = -1e9
_TINY = np.float32(np.finfo(np.float32).tiny)
_NLO = np.float32(np.nextafter(np.float32(-1.0), np.float32(0.0)))
_NSCALE = np.float32(np.float32(1.0) - _NLO)
_SQRT2 = np.float32(np.sqrt(2.0))


def _np_threefry2x32(k1, k2, x0, x1):
    rot_a = (13, 15, 26, 6)
    rot_b = (17, 29, 16, 24)
    m = np.uint64(0xFFFFFFFF)

    def add(a, b):
        return np.uint32((np.uint64(a) + np.uint64(b)) & m)

    def rotl(x, r):
        x = int(x)
        return np.uint32(((x << r) | (x >> (32 - r))) & 0xFFFFFFFF)

    ks = (np.uint32(k1), np.uint32(k2),
          np.uint32(k1) ^ np.uint32(k2) ^ np.uint32(0x1BD11BDA))
    x0, x1 = add(x0, ks[0]), add(x1, ks[1])
    for i, rots in enumerate((rot_a, rot_b, rot_a, rot_b, rot_a)):
        for r in rots:
            x0 = add(x0, x1)
            x1 = rotl(x1, r) ^ x0
        x0 = add(x0, ks[(i + 1) % 3])
        x1 = add(add(x1, ks[(i + 2) % 3]), np.uint32(i + 1))
    return x0, x1


_KRAND = _np_threefry2x32(np.uint32(0), np.uint32(42),
                          np.uint32(0), np.uint32(0))
_KCAT = _np_threefry2x32(np.uint32(0), np.uint32(42),
                         np.uint32(0), np.uint32(1))


def _random_bits(j, keypair):
    k1 = jnp.uint32(keypair[0])
    k2 = jnp.uint32(keypair[1])
    ks2 = jnp.uint32(int(keypair[0] ^ keypair[1] ^ np.uint32(0x1BD11BDA)))
    ks = (k1, k2, ks2)
    x0 = jnp.full(j.shape, k1, jnp.uint32)
    x1 = j + k2
    rot_a = (13, 15, 26, 6)
    rot_b = (17, 29, 16, 24)
    for i, rots in enumerate((rot_a, rot_b, rot_a, rot_b, rot_a)):
        for r in rots:
            x0 = x0 + x1
            x1 = ((x1 << jnp.uint32(r)) | (x1 >> jnp.uint32(32 - r))) ^ x0
        x0 = x0 + ks[(i + 1) % 3]
        x1 = x1 + ks[(i + 2) % 3] + jnp.uint32(i + 1)
    return x0 ^ x1


def _bits_to_unit_float(bits):
    fb = (bits >> jnp.uint32(9)) | jnp.uint32(0x3F800000)
    return jax.lax.bitcast_convert_type(fb, jnp.float32) - 1.0


def _erfinv(x):
    w = -jnp.log((1.0 - x) * (1.0 + x))
    wa = w - 2.5
    pa = jnp.float32(2.81022636e-08)
    for c in (3.43273939e-07, -3.5233877e-06, -4.39150654e-06,
              0.00021858087, -0.00125372503, -0.00417768164,
              0.246640727, 1.50140941):
        pa = pa * wa + jnp.float32(c)
    wb = jnp.sqrt(jnp.maximum(w, 1e-6)) - 3.0
    pb = jnp.float32(-0.000200214257)
    for c in (0.000100950558, 0.00134934322, -0.00367342844,
              0.00573950773, -0.0076224613, 0.00943887047,
              1.00167406, 2.83297682):
        pb = pb * wb + jnp.float32(c)
    return jnp.where(w < 5.0, pa, pb) * x


def _fused_kernel(xt_ref, w1_ref, b1_ref, w2_ref, b2_ref,
                  w3_ref, b3_ref, pw_ref, pb_ref, out_ref):
    h = jnp.maximum(
        jnp.dot(w1_ref[...], xt_ref[...], preferred_element_type=jnp.float32)
        + b1_ref[...], 0.0)
    h = jnp.maximum(
        jnp.dot(w2_ref[...], h, preferred_element_type=jnp.float32)
        + b2_ref[...], 0.0)
    h = jnp.maximum(
        jnp.dot(w3_ref[...], h, preferred_element_type=jnp.float32)
        + b3_ref[...], 0.0)
    allv = (jnp.dot(pw_ref[...], h, preferred_element_type=jnp.float32)
            + pb_ref[...])
    mu = allv[0:128]
    sig = jnp.abs(allv[128:256])
    pai = jnp.abs(allv[256:384])

    t = pai.shape[1]
    rr = jax.lax.broadcasted_iota(jnp.int32, (100, t), 0)
    bb = jax.lax.broadcasted_iota(jnp.int32, (100, t), 1) \
        + pl.program_id(0) * t
    jidx = bb * 100 + rr
    bits = _random_bits(jidx.astype(jnp.uint32), _KCAT)
    f = _bits_to_unit_float(bits)
    u = f + _TINY
    g100 = -jnp.log(-jnp.log(u))
    pad7 = jnp.full((7, t), _NEG, jnp.float32)
    g = jnp.concatenate(
        [g100[0:25], pad7, g100[25:50], pad7,
         g100[50:75], pad7, g100[75:100], pad7], axis=0)

    rr8 = jax.lax.broadcasted_iota(jnp.int32, (8, t), 0)
    bb8 = jax.lax.broadcasted_iota(jnp.int32, (8, t), 1) \
        + pl.program_id(0) * t
    jn8 = bb8 * 4 + rr8
    nbits = _random_bits(jn8.astype(jnp.uint32), _KRAND)
    un = _bits_to_unit_float(nbits) * _NSCALE + _NLO
    nrm = _SQRT2 * _erfinv(un)

    z = jnp.log(pai + 1e-20) + g
    sel_mu, sel_sig = [], []
    cidx = jax.lax.broadcasted_iota(jnp.int32, (32, t), 0)
    for k in range(4):
        zk = z[k * 32:(k + 1) * 32]
        m = jnp.max(zk, axis=0, keepdims=True)
        idx = jnp.min(jnp.where(zk == m, cidx, 32), axis=0, keepdims=True)
        onehot = (cidx == idx).astype(jnp.float32)
        sel_mu.append(jnp.sum(onehot * mu[k * 32:(k + 1) * 32],
                              axis=0, keepdims=True))
        sel_sig.append(jnp.sum(onehot * sig[k * 32:(k + 1) * 32],
                               axis=0, keepdims=True))
    outv = nrm[0:4, :] * jnp.concatenate(sel_sig, 0) \
        + jnp.concatenate(sel_mu, 0)
    out_ref[0:4, :] = outv


def kernel(x0, W1, b1, W2, b2, W3, b3, PW, Pb):
    B = x0.shape[0]
    xt = jnp.zeros((8, B), jnp.float32).at[:3].set(x0.T)

    w1 = jnp.zeros((128, 8), jnp.float32).at[:, :3].set(W1.T)
    w2 = W2.T
    w3 = W3.T

    def _heads(j):
        wt = jnp.transpose(PW[j::3], (2, 0, 1))
        wt = jnp.zeros((4, 32, 200), jnp.float32).at[:, :25].set(wt)
        bt = jnp.zeros((4, 32), jnp.float32).at[:, :25].set(Pb[j::3].T)
        return wt.reshape(128, 200), bt.reshape(128, 1)

    wmu, bmu = _heads(0)
    wsig, bsig = _heads(1)
    wpai, bpai = _heads(2)
    pw = jnp.concatenate([wmu, wsig, wpai], 0)
    pb = jnp.concatenate([bmu, bsig, bpai], 0)

    out = pl.pallas_call(
        _fused_kernel,
        grid=(B // _TBLK,),
        in_specs=[
            pl.BlockSpec((8, _TBLK), lambda i: (0, i)),
            pl.BlockSpec((128, 8), lambda i: (0, 0)),
            pl.BlockSpec((128, 1), lambda i: (0, 0)),
            pl.BlockSpec((256, 128), lambda i: (0, 0)),
            pl.BlockSpec((256, 1), lambda i: (0, 0)),
            pl.BlockSpec((200, 256), lambda i: (0, 0)),
            pl.BlockSpec((200, 1), lambda i: (0, 0)),
            pl.BlockSpec((384, 200), lambda i: (0, 0)),
            pl.BlockSpec((384, 1), lambda i: (0, 0)),
        ],
        out_specs=pl.BlockSpec((8, _TBLK), lambda i: (0, i)),
        out_shape=jax.ShapeDtypeStruct((8, B), jnp.float32),
    )(xt, w1, b1[:, None], w2, b2[:, None], w3, b3[:, None], pw, pb)
    return out[:4].T


# speedup vs baseline: 1.0129x; 1.0129x over previous
import numpy as np
import jax
import jax.numpy as jnp
from jax.experimental import pallas as pl

_TBLK = 2048
_NEG = -1e9
_TINY = np.float32(np.finfo(np.float32).tiny)
_NLO = np.float32(np.nextafter(np.float32(-1.0), np.float32(0.0)))
_NSCALE = np.float32(np.float32(1.0) - _NLO)
_SQRT2 = np.float32(np.sqrt(2.0))


def _np_threefry2x32(k1, k2, x0, x1):
    rot_a = (13, 15, 26, 6)
    rot_b = (17, 29, 16, 24)
    m = np.uint64(0xFFFFFFFF)

    def add(a, b):
        return np.uint32((np.uint64(a) + np.uint64(b)) & m)

    def rotl(x, r):
        x = int(x)
        return np.uint32(((x << r) | (x >> (32 - r))) & 0xFFFFFFFF)

    ks = (np.uint32(k1), np.uint32(k2),
          np.uint32(k1) ^ np.uint32(k2) ^ np.uint32(0x1BD11BDA))
    x0, x1 = add(x0, ks[0]), add(x1, ks[1])
    for i, rots in enumerate((rot_a, rot_b, rot_a, rot_b, rot_a)):
        for r in rots:
            x0 = add(x0, x1)
            x1 = rotl(x1, r) ^ x0
        x0 = add(x0, ks[(i + 1) % 3])
        x1 = add(add(x1, ks[(i + 2) % 3]), np.uint32(i + 1))
    return x0, x1


_KRAND = _np_threefry2x32(np.uint32(0), np.uint32(42),
                          np.uint32(0), np.uint32(0))
_KCAT = _np_threefry2x32(np.uint32(0), np.uint32(42),
                         np.uint32(0), np.uint32(1))


def _random_bits(j, keypair):
    k1 = jnp.uint32(keypair[0])
    k2 = jnp.uint32(keypair[1])
    ks2 = jnp.uint32(int(keypair[0] ^ keypair[1] ^ np.uint32(0x1BD11BDA)))
    ks = (k1, k2, ks2)
    x0 = jnp.full(j.shape, k1, jnp.uint32)
    x1 = j + k2
    rot_a = (13, 15, 26, 6)
    rot_b = (17, 29, 16, 24)
    for i, rots in enumerate((rot_a, rot_b, rot_a, rot_b, rot_a)):
        for r in rots:
            x0 = x0 + x1
            x1 = ((x1 << jnp.uint32(r)) | (x1 >> jnp.uint32(32 - r))) ^ x0
        x0 = x0 + ks[(i + 1) % 3]
        x1 = x1 + ks[(i + 2) % 3] + jnp.uint32(i + 1)
    return x0 ^ x1


def _bits_to_unit_float(bits):
    fb = (bits >> jnp.uint32(9)) | jnp.uint32(0x3F800000)
    return jax.lax.bitcast_convert_type(fb, jnp.float32) - 1.0


def _erfinv(x):
    w = -jnp.log((1.0 - x) * (1.0 + x))
    wa = w - 2.5
    pa = jnp.float32(2.81022636e-08)
    for c in (3.43273939e-07, -3.5233877e-06, -4.39150654e-06,
              0.00021858087, -0.00125372503, -0.00417768164,
              0.246640727, 1.50140941):
        pa = pa * wa + jnp.float32(c)
    wb = jnp.sqrt(jnp.maximum(w, 1e-6)) - 3.0
    pb = jnp.float32(-0.000200214257)
    for c in (0.000100950558, 0.00134934322, -0.00367342844,
              0.00573950773, -0.0076224613, 0.00943887047,
              1.00167406, 2.83297682):
        pb = pb * wb + jnp.float32(c)
    return jnp.where(w < 5.0, pa, pb) * x


def _fused_kernel(xt_ref, w1_ref, b1_ref, w2_ref, b2_ref,
                  w3_ref, b3_ref, pw_ref, pb_ref, out_ref):
    h = jnp.maximum(
        jnp.dot(w1_ref[...], xt_ref[...], preferred_element_type=jnp.float32)
        + b1_ref[...], 0.0)
    h = jnp.maximum(
        jnp.dot(w2_ref[...], h, preferred_element_type=jnp.float32)
        + b2_ref[...], 0.0)
    h = jnp.maximum(
        jnp.dot(w3_ref[...], h, preferred_element_type=jnp.float32)
        + b3_ref[...], 0.0)
    allv = (jnp.dot(pw_ref[...], h, preferred_element_type=jnp.float32)
            + pb_ref[...])
    mu = allv[0:128]
    sig = jnp.abs(allv[128:256])
    pai = jnp.abs(allv[256:384])

    t = pai.shape[1]
    rr = jax.lax.broadcasted_iota(jnp.int32, (100, t), 0)
    bb = jax.lax.broadcasted_iota(jnp.int32, (100, t), 1) \
        + pl.program_id(0) * t
    jidx = bb * 100 + rr
    bits = _random_bits(jidx.astype(jnp.uint32), _KCAT)
    f = _bits_to_unit_float(bits)
    u = f + _TINY
    g100 = -jnp.log(-jnp.log(u))
    pad7 = jnp.full((7, t), _NEG, jnp.float32)
    g = jnp.concatenate(
        [g100[0:25], pad7, g100[25:50], pad7,
         g100[50:75], pad7, g100[75:100], pad7], axis=0)

    rr8 = jax.lax.broadcasted_iota(jnp.int32, (8, t), 0)
    bb8 = jax.lax.broadcasted_iota(jnp.int32, (8, t), 1) \
        + pl.program_id(0) * t
    jn8 = bb8 * 4 + rr8
    nbits = _random_bits(jn8.astype(jnp.uint32), _KRAND)
    un = _bits_to_unit_float(nbits) * _NSCALE + _NLO
    nrm = _SQRT2 * _erfinv(un)

    z = jnp.log(pai + 1e-20) + g
    sel_mu, sel_sig = [], []
    cidx = jax.lax.broadcasted_iota(jnp.int32, (32, t), 0)
    for k in range(4):
        zk = z[k * 32:(k + 1) * 32]
        idx = jnp.argmax(zk, axis=0)[None, :]
        onehot = (cidx == idx).astype(jnp.float32)
        sel_mu.append(jnp.sum(onehot * mu[k * 32:(k + 1) * 32],
                              axis=0, keepdims=True))
        sel_sig.append(jnp.sum(onehot * sig[k * 32:(k + 1) * 32],
                               axis=0, keepdims=True))
    outv = nrm[0:4, :] * jnp.concatenate(sel_sig, 0) \
        + jnp.concatenate(sel_mu, 0)
    out_ref[0:4, :] = outv


def kernel(x0, W1, b1, W2, b2, W3, b3, PW, Pb):
    B = x0.shape[0]
    xt = jnp.zeros((8, B), jnp.float32).at[:3].set(x0.T)

    w1 = jnp.zeros((128, 8), jnp.float32).at[:, :3].set(W1.T)
    w2 = W2.T
    w3 = W3.T

    def _heads(j):
        wt = jnp.transpose(PW[j::3], (2, 0, 1))
        wt = jnp.zeros((4, 32, 200), jnp.float32).at[:, :25].set(wt)
        bt = jnp.zeros((4, 32), jnp.float32).at[:, :25].set(Pb[j::3].T)
        return wt.reshape(128, 200), bt.reshape(128, 1)

    wmu, bmu = _heads(0)
    wsig, bsig = _heads(1)
    wpai, bpai = _heads(2)
    pw = jnp.concatenate([wmu, wsig, wpai], 0)
    pb = jnp.concatenate([bmu, bsig, bpai], 0)

    out = pl.pallas_call(
        _fused_kernel,
        grid=(B // _TBLK,),
        in_specs=[
            pl.BlockSpec((8, _TBLK), lambda i: (0, i)),
            pl.BlockSpec((128, 8), lambda i: (0, 0)),
            pl.BlockSpec((128, 1), lambda i: (0, 0)),
            pl.BlockSpec((256, 128), lambda i: (0, 0)),
            pl.BlockSpec((256, 1), lambda i: (0, 0)),
            pl.BlockSpec((200, 256), lambda i: (0, 0)),
            pl.BlockSpec((200, 1), lambda i: (0, 0)),
            pl.BlockSpec((384, 200), lambda i: (0, 0)),
            pl.BlockSpec((384, 1), lambda i: (0, 0)),
        ],
        out_specs=pl.BlockSpec((8, _TBLK), lambda i: (0, i)),
        out_shape=jax.ShapeDtypeStruct((8, B), jnp.float32),
    )(xt, w1, b1[:, None], w2, b2[:, None], w3, b3[:, None], pw, pb)
    return out[:4].T


# speedup vs baseline: 1.0148x; 1.0019x over previous
import numpy as np
import jax
import jax.numpy as jnp
from jax.experimental import pallas as pl

_TBLK = 2048
_NEG = -1e9
_TINY = np.float32(np.finfo(np.float32).tiny)
_NLO = np.float32(np.nextafter(np.float32(-1.0), np.float32(0.0)))
_NSCALE = np.float32(np.float32(1.0) - _NLO)
_SQRT2 = np.float32(np.sqrt(2.0))


def _np_threefry2x32(k1, k2, x0, x1):
    rot_a = (13, 15, 26, 6)
    rot_b = (17, 29, 16, 24)
    m = np.uint64(0xFFFFFFFF)

    def add(a, b):
        return np.uint32((np.uint64(a) + np.uint64(b)) & m)

    def rotl(x, r):
        x = int(x)
        return np.uint32(((x << r) | (x >> (32 - r))) & 0xFFFFFFFF)

    ks = (np.uint32(k1), np.uint32(k2),
          np.uint32(k1) ^ np.uint32(k2) ^ np.uint32(0x1BD11BDA))
    x0, x1 = add(x0, ks[0]), add(x1, ks[1])
    for i, rots in enumerate((rot_a, rot_b, rot_a, rot_b, rot_a)):
        for r in rots:
            x0 = add(x0, x1)
            x1 = rotl(x1, r) ^ x0
        x0 = add(x0, ks[(i + 1) % 3])
        x1 = add(add(x1, ks[(i + 2) % 3]), np.uint32(i + 1))
    return x0, x1


_KRAND = _np_threefry2x32(np.uint32(0), np.uint32(42),
                          np.uint32(0), np.uint32(0))
_KCAT = _np_threefry2x32(np.uint32(0), np.uint32(42),
                         np.uint32(0), np.uint32(1))


def _random_bits(j, keypair):
    k1 = jnp.uint32(keypair[0])
    k2 = jnp.uint32(keypair[1])
    ks2 = jnp.uint32(int(keypair[0] ^ keypair[1] ^ np.uint32(0x1BD11BDA)))
    ks = (k1, k2, ks2)
    x0 = jnp.full(j.shape, k1, jnp.uint32)
    x1 = j + k2
    rot_a = (13, 15, 26, 6)
    rot_b = (17, 29, 16, 24)
    for i, rots in enumerate((rot_a, rot_b, rot_a, rot_b, rot_a)):
        for r in rots:
            x0 = x0 + x1
            x1 = ((x1 << jnp.uint32(r)) | (x1 >> jnp.uint32(32 - r))) ^ x0
        x0 = x0 + ks[(i + 1) % 3]
        x1 = x1 + ks[(i + 2) % 3] + jnp.uint32(i + 1)
    return x0 ^ x1


def _bits_to_unit_float(bits):
    fb = (bits >> jnp.uint32(9)) | jnp.uint32(0x3F800000)
    return jax.lax.bitcast_convert_type(fb, jnp.float32) - 1.0


def _erfinv(x):
    w = -jnp.log((1.0 - x) * (1.0 + x))
    wa = w - 2.5
    pa = jnp.float32(2.81022636e-08)
    for c in (3.43273939e-07, -3.5233877e-06, -4.39150654e-06,
              0.00021858087, -0.00125372503, -0.00417768164,
              0.246640727, 1.50140941):
        pa = pa * wa + jnp.float32(c)
    wb = jnp.sqrt(jnp.maximum(w, 1e-6)) - 3.0
    pb = jnp.float32(-0.000200214257)
    for c in (0.000100950558, 0.00134934322, -0.00367342844,
              0.00573950773, -0.0076224613, 0.00943887047,
              1.00167406, 2.83297682):
        pb = pb * wb + jnp.float32(c)
    return jnp.where(w < 5.0, pa, pb) * x


def _fused_kernel(xt_ref, w1_ref, b1_ref, w2_ref, b2_ref,
                  w3_ref, b3_ref, pw_ref, pb_ref, out_ref):
    h = jnp.maximum(
        jnp.dot(w1_ref[...], xt_ref[...], preferred_element_type=jnp.float32)
        + b1_ref[...], 0.0)
    h = jnp.maximum(
        jnp.dot(w2_ref[...], h, preferred_element_type=jnp.float32)
        + b2_ref[...], 0.0)
    h = jnp.maximum(
        jnp.dot(w3_ref[...], h, preferred_element_type=jnp.float32)
        + b3_ref[...], 0.0)
    allv = (jnp.dot(pw_ref[...], h, preferred_element_type=jnp.float32)
            + pb_ref[...])
    mu = allv[0:128]
    sig = allv[128:256]
    pai = jnp.abs(allv[256:384])

    t = pai.shape[1]
    rr = jax.lax.broadcasted_iota(jnp.int32, (100, t), 0)
    bb = jax.lax.broadcasted_iota(jnp.int32, (100, t), 1) \
        + pl.program_id(0) * t
    jidx = bb * 100 + rr
    bits = _random_bits(jidx.astype(jnp.uint32), _KCAT)
    f = _bits_to_unit_float(bits)
    u = f + _TINY
    g100 = -jnp.log(-jnp.log(u))
    pad7 = jnp.full((7, t), _NEG, jnp.float32)
    g = jnp.concatenate(
        [g100[0:25], pad7, g100[25:50], pad7,
         g100[50:75], pad7, g100[75:100], pad7], axis=0)

    rr8 = jax.lax.broadcasted_iota(jnp.int32, (8, t), 0)
    bb8 = jax.lax.broadcasted_iota(jnp.int32, (8, t), 1) \
        + pl.program_id(0) * t
    jn8 = bb8 * 4 + rr8
    nbits = _random_bits(jn8.astype(jnp.uint32), _KRAND)
    un = _bits_to_unit_float(nbits) * _NSCALE + _NLO
    nrm = _SQRT2 * _erfinv(un)

    z = jnp.log(pai + 1e-20) + g
    sel_mu, sel_sig = [], []
    cidx = jax.lax.broadcasted_iota(jnp.int32, (32, t), 0)
    for k in range(4):
        zk = z[k * 32:(k + 1) * 32]
        idx = jnp.argmax(zk, axis=0)[None, :]
        onehot = (cidx == idx).astype(jnp.float32)
        sel_mu.append(jnp.sum(onehot * mu[k * 32:(k + 1) * 32],
                              axis=0, keepdims=True))
        sel_sig.append(jnp.sum(onehot * sig[k * 32:(k + 1) * 32],
                               axis=0, keepdims=True))
    outv = nrm[0:4, :] * jnp.abs(jnp.concatenate(sel_sig, 0)) \
        + jnp.concatenate(sel_mu, 0)
    out_ref[0:4, :] = outv


def kernel(x0, W1, b1, W2, b2, W3, b3, PW, Pb):
    B = x0.shape[0]
    xt = jnp.zeros((8, B), jnp.float32).at[:3].set(x0.T)

    w1 = jnp.zeros((128, 8), jnp.float32).at[:, :3].set(W1.T)
    w2 = W2.T
    w3 = W3.T

    def _heads(j):
        wt = jnp.transpose(PW[j::3], (2, 0, 1))
        wt = jnp.zeros((4, 32, 200), jnp.float32).at[:, :25].set(wt)
        bt = jnp.zeros((4, 32), jnp.float32).at[:, :25].set(Pb[j::3].T)
        return wt.reshape(128, 200), bt.reshape(128, 1)

    wmu, bmu = _heads(0)
    wsig, bsig = _heads(1)
    wpai, bpai = _heads(2)
    pw = jnp.concatenate([wmu, wsig, wpai], 0)
    pb = jnp.concatenate([bmu, bsig, bpai], 0)

    out = pl.pallas_call(
        _fused_kernel,
        grid=(B // _TBLK,),
        in_specs=[
            pl.BlockSpec((8, _TBLK), lambda i: (0, i)),
            pl.BlockSpec((128, 8), lambda i: (0, 0)),
            pl.BlockSpec((128, 1), lambda i: (0, 0)),
            pl.BlockSpec((256, 128), lambda i: (0, 0)),
            pl.BlockSpec((256, 1), lambda i: (0, 0)),
            pl.BlockSpec((200, 256), lambda i: (0, 0)),
            pl.BlockSpec((200, 1), lambda i: (0, 0)),
            pl.BlockSpec((384, 200), lambda i: (0, 0)),
            pl.BlockSpec((384, 1), lambda i: (0, 0)),
        ],
        out_specs=pl.BlockSpec((8, _TBLK), lambda i: (0, i)),
        out_shape=jax.ShapeDtypeStruct((8, B), jnp.float32),
    )(xt, w1, b1[:, None], w2, b2[:, None], w3, b3[:, None], pw, pb)
    return out[:4].T
